# baseline (speedup 1.0000x reference)
import functools

import jax
import jax.numpy as jnp
from jax.experimental import pallas as pl
from jax.experimental.pallas import tpu as pltpu


def _gate_kernel(x_ref, w1t_ref, b1_ref, w2_ref, b2_ref, o_ref, scale_ref,
                 *, inv_s, s_half):
    s = pl.program_id(1)

    @pl.when(s == 0)
    def _():
        xt = x_ref[0]
        ssum = jnp.sum(xt, axis=0, keepdims=True)
        smax = jnp.max(xt, axis=0, keepdims=True)
        pools = jnp.concatenate([ssum * inv_s, smax], axis=0)
        h = jax.lax.dot_general(pools, w1t_ref[...], (((1,), (1,)), ((), ())),
                                preferred_element_type=jnp.float32)
        h = jnp.maximum(h + b1_ref[...], 0.0)
        att2 = jnp.dot(h, w2_ref[...],
                       preferred_element_type=jnp.float32)
        att = att2[0:1, :] + att2[1:2, :] + 2.0 * b2_ref[...]
        scale_ref[...] = jax.nn.sigmoid(att)

    half = x_ref[0, pl.ds(s * s_half, s_half), :]
    o_ref[0] = (half * scale_ref[...]).astype(o_ref.dtype)


def kernel(x, w1, b1, w2, b2):
    B, C, D, H, W = x.shape
    S = D * H * W
    xs = x.transpose(0, 2, 3, 4, 1).reshape(B, S, C)
    w1t = w1.T
    b1r = b1.astype(jnp.float32).reshape(1, -1)
    b2r = b2.astype(jnp.float32).reshape(1, -1)
    Cr = w1.shape[1]
    s_half = S // 2

    body = functools.partial(_gate_kernel, inv_s=1.0 / S, s_half=s_half)

    itemsize = jnp.dtype(x.dtype).itemsize
    cost = pl.CostEstimate(
        flops=4 * B * C * S,
        transcendentals=B * C,
        bytes_accessed=2 * B * C * S * itemsize)

    out = pl.pallas_call(
        body,
        out_shape=jax.ShapeDtypeStruct((B, S, C), x.dtype),
        grid=(B, 2),
        in_specs=[
            pl.BlockSpec((1, S, C), lambda b, s: (b, 0, 0)),
            pl.BlockSpec((Cr, C), lambda b, s: (0, 0)),
            pl.BlockSpec((1, Cr), lambda b, s: (0, 0)),
            pl.BlockSpec((Cr, C), lambda b, s: (0, 0)),
            pl.BlockSpec((1, C), lambda b, s: (0, 0)),
        ],
        out_specs=pl.BlockSpec((1, s_half, C), lambda b, s: (b, s, 0)),
        scratch_shapes=[pltpu.VMEM((1, C), jnp.float32)],
        compiler_params=pltpu.CompilerParams(
            dimension_semantics=("parallel", "arbitrary")),
        cost_estimate=cost,
    )(xs, w1t, b1r, w2, b2r)

    return out.reshape(B, D, H, W, C).transpose(0, 4, 1, 2, 3)


# speedup vs baseline: 1.6217x; 1.6217x over previous
import functools

import jax
import jax.numpy as jnp
from jax.experimental import pallas as pl
from jax.experimental.pallas import tpu as pltpu


def _gate_kernel(x_ref, w1t_ref, b1_ref, w2_ref, b2_ref, o_ref, *, inv_s):
    xt = x_ref[0]
    ssum = jnp.sum(xt, axis=0, keepdims=True)
    smax = jnp.max(xt, axis=0, keepdims=True)
    pools = jnp.concatenate([ssum * inv_s, smax], axis=0)
    h = jax.lax.dot_general(pools, w1t_ref[...], (((1,), (1,)), ((), ())),
                            preferred_element_type=jnp.float32) + b1_ref[...]
    h = jnp.maximum(h, 0.0)
    att2 = jnp.dot(h, w2_ref[...],
                   preferred_element_type=jnp.float32)
    att = att2[0:1, :] + att2[1:2, :] + 2.0 * b2_ref[...]
    scale = jax.nn.sigmoid(att)
    o_ref[0] = (xt * scale).astype(o_ref.dtype)


def kernel(x, w1, b1, w2, b2):
    B, C, D, H, W = x.shape
    S = D * H * W
    xs = x.transpose(0, 2, 3, 4, 1).reshape(B, S, C)
    w1t = w1.T
    b1r = b1.astype(jnp.float32).reshape(1, -1)
    b2r = b2.astype(jnp.float32).reshape(1, -1)
    Cr = w1.shape[1]

    body = functools.partial(_gate_kernel, inv_s=1.0 / S)

    itemsize = jnp.dtype(x.dtype).itemsize
    cost = pl.CostEstimate(
        flops=4 * B * C * S,
        transcendentals=B * C,
        bytes_accessed=2 * B * C * S * itemsize)

    out = pl.pallas_call(
        body,
        out_shape=jax.ShapeDtypeStruct((B, S, C), x.dtype),
        grid=(B,),
        in_specs=[
            pl.BlockSpec((1, S, C), lambda b: (b, 0, 0)),
            pl.BlockSpec((Cr, C), lambda b: (0, 0)),
            pl.BlockSpec((1, Cr), lambda b: (0, 0)),
            pl.BlockSpec((Cr, C), lambda b: (0, 0)),
            pl.BlockSpec((1, C), lambda b: (0, 0)),
        ],
        out_specs=pl.BlockSpec((1, S, C), lambda b: (b, 0, 0)),
        compiler_params=pltpu.CompilerParams(
            dimension_semantics=("parallel",)),
        cost_estimate=cost,
    )(xs, w1t, b1r, w2, b2r)

    return out.reshape(B, D, H, W, C).transpose(0, 4, 1, 2, 3)
